# baseline (speedup 1.0000x reference)
import jax
import jax.numpy as jnp
from jax import lax
from jax.experimental import pallas as pl
from jax.experimental.pallas import tpu as pltpu
from jax.experimental.pallas import tpu_sc as plsc

_N = 10000
_E = 320000
_D = 128
_H = 8
_DH = 16
_DFF = 512
_CLAMP = 5.0

_HH = _H // 2
_HD = _HH * _DH
_CW = _HD + 8

_NP = 10112
_EPT = _E // 16
_C = 125
_NCHUNK = _EPT // _C
_NPAIR = _NCHUNK // 2
_RPT = _NP // 16

_BLK = 1000
_GRID = _N // _BLK

_DN_T = (((1,), (1,)), ((), ()))
_F32 = jnp.float32


def _perm_mat(n):
    r = lax.broadcasted_iota(jnp.int32, (n, n), 0)
    d = lax.broadcasted_iota(jnp.int32, (n, n), 1)
    src = (d // 32) * 32 + (d % 2) * _DH + (d % 32) // 2
    return (r == src).astype(_F32)


def _proj_body(feat_ref, wq_ref, wk_ref, wv_ref,
               q0_ref, q1_ref, kv0_ref, kv1_ref):
    x = feat_ref[...]
    qn = lax.dot_general(x, wq_ref[...], _DN_T, preferred_element_type=_F32)
    kn = lax.dot_general(x, wk_ref[...], _DN_T, preferred_element_type=_F32)
    vn = lax.dot_general(x, wv_ref[...], _DN_T, preferred_element_type=_F32)
    dn = (((1,), (0,)), ((), ()))
    p = _perm_mat(_HD)
    bf = jnp.bfloat16
    q0_ref[...] = lax.dot_general(qn[:, :_HD], p, dn,
                                  preferred_element_type=_F32).astype(bf)
    q1_ref[...] = lax.dot_general(qn[:, _HD:], p, dn,
                                  preferred_element_type=_F32).astype(bf)
    kv0_ref[...] = jnp.concatenate(
        [lax.dot_general(kn[:, :_HD], p, dn,
                         preferred_element_type=_F32).astype(bf),
         lax.dot_general(vn[:, :_HD], p, dn,
                         preferred_element_type=_F32).astype(bf)], axis=1)
    kv1_ref[...] = jnp.concatenate(
        [lax.dot_general(kn[:, _HD:], p, dn,
                         preferred_element_type=_F32).astype(bf),
         lax.dot_general(vn[:, _HD:], p, dn,
                         preferred_element_type=_F32).astype(bf)], axis=1)


def _proj(feat, Wq, Wk, Wv):
    return pl.pallas_call(
        _proj_body,
        grid=(_GRID,),
        in_specs=[
            pl.BlockSpec((_BLK, _D), lambda i: (i, 0)),
            pl.BlockSpec((_D, _D), lambda i: (0, 0)),
            pl.BlockSpec((_D, _D), lambda i: (0, 0)),
            pl.BlockSpec((_D, _D), lambda i: (0, 0)),
        ],
        out_specs=[
            pl.BlockSpec((_BLK, _HD), lambda i: (i, 0)),
            pl.BlockSpec((_BLK, _HD), lambda i: (i, 0)),
            pl.BlockSpec((_BLK, _D), lambda i: (i, 0)),
            pl.BlockSpec((_BLK, _D), lambda i: (i, 0)),
        ],
        out_shape=[
            jax.ShapeDtypeStruct((_N, _HD), jnp.bfloat16),
            jax.ShapeDtypeStruct((_N, _HD), jnp.bfloat16),
            jax.ShapeDtypeStruct((_N, _D), jnp.bfloat16),
            jax.ShapeDtypeStruct((_N, _D), jnp.bfloat16),
        ],
    )(feat, Wq, Wk, Wv)


def _sc_body(src3_hbm, dst3_hbm, q0_hbm, q1_hbm, kv0_hbm, kv1_hbm, z_hbm,
             out_hbm,
             sidx3, didx3, qbuf_a, kvbuf_a, comb_a, qbuf_b, kvbuf_b, comb_b,
             acc, sem_qa, sem_kva, sem_qb, sem_kvb, sem_sa, sem_sb):
    c = lax.axis_index("c")
    s = lax.axis_index("s")

    r0 = s * _RPT
    pltpu.sync_copy(z_hbm.at[pl.ds(r0, _RPT)], acc.at[pl.ds(r0, _RPT)])

    cb = s * _NCHUNK
    pltpu.sync_copy(src3_hbm.at[pl.ds(cb, _NCHUNK)], sidx3)
    pltpu.sync_copy(dst3_hbm.at[pl.ds(cb, _NCHUNK)], didx3)

    plsc.subcore_barrier()

    lanes = lax.iota(jnp.int32, 16)

    def _gather(ci, qbuf, kvbuf, sq, skv):
        @pl.when(c == 0)
        def _():
            pltpu.async_copy(q0_hbm.at[didx3.at[ci, 0]], qbuf, sq)
            pltpu.async_copy(kv0_hbm.at[sidx3.at[ci, 0]], kvbuf, skv)

        @pl.when(c == 1)
        def _():
            pltpu.async_copy(q1_hbm.at[didx3.at[ci, 0]], qbuf, sq)
            pltpu.async_copy(kv1_hbm.at[sidx3.at[ci, 0]], kvbuf, skv)

    def _wait_gather(qbuf, kvbuf, sq, skv):
        pltpu.make_async_copy(q0_hbm.at[pl.ds(0, _C)], qbuf, sq).wait()
        pltpu.make_async_copy(kv0_hbm.at[pl.ds(0, _C)], kvbuf, skv).wait()

    def _scat(ci, comb, ss):
        pltpu.async_copy(comb, acc.at[didx3.at[ci, 0]], ss, add=True)

    def _wait_scat(comb, ss):
        pltpu.make_async_copy(z_hbm.at[pl.ds(0, _C)], comb, ss).wait()

    perm8 = lanes ^ 8
    perm4 = lanes ^ 4
    perm2 = lanes ^ 2
    perm1 = lanes ^ 1
    lane_grp = lanes // 4
    bcast_idx = [jnp.full((16,), 4 * h, jnp.int32) for h in range(_HH)]

    def _perm(x, pidx):
        return x.at[pidx].get(mode="promise_in_bounds")

    def _compute(qbuf, kvbuf, comb):
        @plsc.parallel_loop(0, _C, unroll=25)
        def _edge(e):
            parts = []
            vs = []
            for hp in range(_HH // 2):
                qp = plsc.unpack(qbuf[e, pl.ds(hp * 32, 32)],
                                 format=plsc.PackFormat.INTERLEAVED)
                kp = plsc.unpack(kvbuf[e, pl.ds(hp * 32, 32)],
                                 format=plsc.PackFormat.INTERLEAVED)
                vp = plsc.unpack(kvbuf[e, pl.ds(_HD + hp * 32, 32)],
                                 format=plsc.PackFormat.INTERLEAVED)
                vs += [vp[0], vp[1]]
                for j in range(2):
                    p = qp[j] * kp[j]
                    p = p + _perm(p, perm8)
                    p = p + _perm(p, perm4)
                    parts.append(p)
            m = parts[0]
            for h in range(1, _HH):
                m = jnp.where(lane_grp == h, parts[h], m)
            m = m + _perm(m, perm2)
            m = m + _perm(m, perm1)
            u4 = jnp.clip(m * 0.25, -_CLAMP, _CLAMP)
            w4 = jnp.exp(u4)
            for h in range(_HH):
                w = _perm(w4, bcast_idx[h])
                comb[e, pl.ds(h * _DH, _DH)] = w * vs[h]
            wsel = jnp.where(lanes < _HH, _perm(w4, (lanes % 4) * 4), 0.0)
            plsc.store_scatter(comb,
                               [jnp.full((16,), e, jnp.int32), _HD + lanes],
                               wsel, mask=lanes < 8)

    _gather(0, qbuf_a, kvbuf_a, sem_qa, sem_kva)

    def _pair(i, carry):
        ca = 2 * i
        _gather(ca + 1, qbuf_b, kvbuf_b, sem_qb, sem_kvb)

        @pl.when(i > 0)
        def _():
            _wait_scat(comb_a, sem_sa)

        _wait_gather(qbuf_a, kvbuf_a, sem_qa, sem_kva)
        _compute(qbuf_a, kvbuf_a, comb_a)
        _scat(ca, comb_a, sem_sa)

        @pl.when(i < _NPAIR - 1)
        def _():
            _gather(ca + 2, qbuf_a, kvbuf_a, sem_qa, sem_kva)

        @pl.when(i > 0)
        def _():
            _wait_scat(comb_b, sem_sb)

        _wait_gather(qbuf_b, kvbuf_b, sem_qb, sem_kvb)
        _compute(qbuf_b, kvbuf_b, comb_b)
        _scat(ca + 1, comb_b, sem_sb)
        return carry

    lax.fori_loop(0, _NPAIR, _pair, 0)

    _wait_scat(comb_a, sem_sa)
    _wait_scat(comb_b, sem_sb)

    plsc.subcore_barrier()
    pltpu.sync_copy(acc.at[pl.ds(r0, _RPT)], out_hbm.at[c, pl.ds(r0, _RPT)])


def _sc_edge(src3, dst3, q0, q1, kv0, kv1, z):
    mesh = plsc.VectorSubcoreMesh(core_axis_name="c", subcore_axis_name="s")
    return pl.kernel(
        _sc_body,
        mesh=mesh,
        compiler_params=pltpu.CompilerParams(needs_layout_passes=False,
                                             use_tc_tiling_on_sc=False),
        out_type=[
            jax.ShapeDtypeStruct((2, _NP, _CW), _F32),
        ],
        scratch_types=[
            pltpu.VMEM((_NCHUNK, 1, _C), jnp.int32),
            pltpu.VMEM((_NCHUNK, 1, _C), jnp.int32),
            pltpu.VMEM((_C, _HD), jnp.bfloat16),
            pltpu.VMEM((_C, _D), jnp.bfloat16),
            pltpu.VMEM((_C, _CW), _F32),
            pltpu.VMEM((_C, _HD), jnp.bfloat16),
            pltpu.VMEM((_C, _D), jnp.bfloat16),
            pltpu.VMEM((_C, _CW), _F32),
            pltpu.VMEM_SHARED((_NP, _CW), _F32),
            pltpu.SemaphoreType.DMA,
            pltpu.SemaphoreType.DMA,
            pltpu.SemaphoreType.DMA,
            pltpu.SemaphoreType.DMA,
            pltpu.SemaphoreType.DMA,
            pltpu.SemaphoreType.DMA,
        ],
    )(src3, dst3, q0, q1, kv0, kv1, z)


def _epi_body(a_ref, feat_ref, wo_ref, g1_ref, bt1_ref, w1_ref,
              bb1_ref, w2_ref, bb2_ref, g2_ref, bt2_ref, out_ref):
    a0 = a_ref[0]
    a1 = a_ref[1]
    num = jnp.concatenate([a0[:, :_HD], a1[:, :_HD]], axis=1)
    den = jnp.concatenate([a0[:, _HD:_HD + _HH],
                           a1[:, _HD:_HD + _HH]], axis=1)
    r = lax.broadcasted_iota(jnp.int32, (_H, _D), 0)
    cc = lax.broadcasted_iota(jnp.int32, (_H, _D), 1)
    sel = (cc // _DH == r).astype(_F32)
    den_e = lax.dot_general(den, sel, (((1,), (0,)), ((), ())),
                            preferred_element_type=_F32)
    den_e = jnp.where(den_e == 0.0, 1.0, den_e)
    agg = num / den_e
    uh = lax.dot_general(agg, wo_ref[...], _DN_T, preferred_element_type=_F32)
    x1 = feat_ref[...] + uh
    mu = jnp.mean(x1, axis=-1, keepdims=True)
    var = jnp.mean((x1 - mu) ** 2, axis=-1, keepdims=True)
    h1 = (x1 - mu) / jnp.sqrt(var + 1e-5) * g1_ref[...] + bt1_ref[...]
    t = jnp.maximum(
        lax.dot_general(h1, w1_ref[...], _DN_T, preferred_element_type=_F32)
        + bb1_ref[...], 0.0)
    f = lax.dot_general(t, w2_ref[...], _DN_T,
                        preferred_element_type=_F32) + bb2_ref[...]
    x2 = h1 + f
    mu2 = jnp.mean(x2, axis=-1, keepdims=True)
    var2 = jnp.mean((x2 - mu2) ** 2, axis=-1, keepdims=True)
    out_ref[...] = (x2 - mu2) / jnp.sqrt(var2 + 1e-5) * g2_ref[...] \
        + bt2_ref[...]


def _epi(a, feat, Wo, ln1_g, ln1_b, W1, b1, W2, b2, ln2_g, ln2_b):
    full = lambda shape: pl.BlockSpec(shape, lambda i: tuple(0 for _ in shape))
    return pl.pallas_call(
        _epi_body,
        grid=(_GRID,),
        in_specs=[
            pl.BlockSpec((2, _BLK, _CW), lambda i: (0, i, 0)),
            pl.BlockSpec((_BLK, _D), lambda i: (i, 0)),
            full((_D, _D)),
            full((_D,)),
            full((_D,)),
            full((_DFF, _D)),
            full((_DFF,)),
            full((_D, _DFF)),
            full((_D,)),
            full((_D,)),
            full((_D,)),
        ],
        out_specs=pl.BlockSpec((_BLK, _D), lambda i: (i, 0)),
        out_shape=jax.ShapeDtypeStruct((_N, _D), _F32),
    )(a, feat, Wo, ln1_g, ln1_b, W1, b1, W2, b2, ln2_g, ln2_b)


def kernel(feat, edge_index, Wq, Wk, Wv, Wo, ln1_g, ln1_b, W1, b1, W2, b2,
           ln2_g, ln2_b):
    src3 = edge_index[0].reshape(_E // _C, 1, _C)
    dst3 = edge_index[1].reshape(_E // _C, 1, _C)
    q0, q1, kv0, kv1 = _proj(feat, Wq, Wk, Wv)
    z = jnp.zeros((_NP, _CW), _F32)
    (acc,) = _sc_edge(src3, dst3, q0, q1, kv0, kv1, z)
    out = _epi(acc, feat, Wo, ln1_g, ln1_b,
               W1, b1, W2, b2, ln2_g, ln2_b)
    return out


# speedup vs baseline: 1.1103x; 1.1103x over previous
import jax
import jax.numpy as jnp
from jax import lax
from jax.experimental import pallas as pl
from jax.experimental.pallas import tpu as pltpu
from jax.experimental.pallas import tpu_sc as plsc

_N = 10000
_E = 320000
_D = 128
_H = 8
_DH = 16
_DFF = 512
_CLAMP = 5.0

_HH = _H // 2
_HD = _HH * _DH
_CW = _HD + 8

_NP = 10112
_EPT = _E // 16
_C = 125
_NCHUNK = _EPT // _C
_NPAIR = _NCHUNK // 2
_RPT = _NP // 16

_BLK = 1000
_GRID = _N // _BLK

_DN_T = (((1,), (1,)), ((), ()))
_F32 = jnp.float32


def _perm_mat(n):
    r = lax.broadcasted_iota(jnp.int32, (n, n), 0)
    d = lax.broadcasted_iota(jnp.int32, (n, n), 1)
    src = (d // 32) * 32 + (d % 2) * _DH + (d % 32) // 2
    return (r == src).astype(_F32)


def _proj_body(feat_ref, wq_ref, wk_ref, wv_ref,
               q0_ref, q1_ref, kv0_ref, kv1_ref):
    x = feat_ref[...]
    qn = lax.dot_general(x, wq_ref[...], _DN_T, preferred_element_type=_F32)
    kn = lax.dot_general(x, wk_ref[...], _DN_T, preferred_element_type=_F32)
    vn = lax.dot_general(x, wv_ref[...], _DN_T, preferred_element_type=_F32)
    dn = (((1,), (0,)), ((), ()))
    p = _perm_mat(_HD)
    bf = jnp.bfloat16
    q0_ref[...] = lax.dot_general(qn[:, :_HD], p, dn,
                                  preferred_element_type=_F32).astype(bf)
    q1_ref[...] = lax.dot_general(qn[:, _HD:], p, dn,
                                  preferred_element_type=_F32).astype(bf)
    kv0_ref[...] = jnp.concatenate(
        [lax.dot_general(kn[:, :_HD], p, dn,
                         preferred_element_type=_F32).astype(bf),
         lax.dot_general(vn[:, :_HD], p, dn,
                         preferred_element_type=_F32).astype(bf)], axis=1)
    kv1_ref[...] = jnp.concatenate(
        [lax.dot_general(kn[:, _HD:], p, dn,
                         preferred_element_type=_F32).astype(bf),
         lax.dot_general(vn[:, _HD:], p, dn,
                         preferred_element_type=_F32).astype(bf)], axis=1)


def _proj(feat, Wq, Wk, Wv):
    return pl.pallas_call(
        _proj_body,
        grid=(_GRID,),
        in_specs=[
            pl.BlockSpec((_BLK, _D), lambda i: (i, 0)),
            pl.BlockSpec((_D, _D), lambda i: (0, 0)),
            pl.BlockSpec((_D, _D), lambda i: (0, 0)),
            pl.BlockSpec((_D, _D), lambda i: (0, 0)),
        ],
        out_specs=[
            pl.BlockSpec((_BLK, _HD), lambda i: (i, 0)),
            pl.BlockSpec((_BLK, _HD), lambda i: (i, 0)),
            pl.BlockSpec((_BLK, _D), lambda i: (i, 0)),
            pl.BlockSpec((_BLK, _D), lambda i: (i, 0)),
        ],
        out_shape=[
            jax.ShapeDtypeStruct((_N, _HD), jnp.bfloat16),
            jax.ShapeDtypeStruct((_N, _HD), jnp.bfloat16),
            jax.ShapeDtypeStruct((_N, _D), jnp.bfloat16),
            jax.ShapeDtypeStruct((_N, _D), jnp.bfloat16),
        ],
    )(feat, Wq, Wk, Wv)


def _sc_body(src3_hbm, dst3_hbm, q0_hbm, q1_hbm, kv0_hbm, kv1_hbm, z_hbm,
             out_hbm,
             sidx3, didx3, qbuf_a, kvbuf_a, comb_a, qbuf_b, kvbuf_b, comb_b,
             acc, sem_qa, sem_kva, sem_qb, sem_kvb, sem_sa, sem_sb):
    c = lax.axis_index("c")
    s = lax.axis_index("s")

    r0 = s * _RPT
    pltpu.sync_copy(z_hbm.at[pl.ds(r0, _RPT)], acc.at[pl.ds(r0, _RPT)])

    cb = s * _NCHUNK
    pltpu.sync_copy(src3_hbm.at[pl.ds(cb, _NCHUNK)], sidx3)
    pltpu.sync_copy(dst3_hbm.at[pl.ds(cb, _NCHUNK)], didx3)

    plsc.subcore_barrier()

    lanes = lax.iota(jnp.int32, 16)

    def _gather(ci, qbuf, kvbuf, sq, skv):
        @pl.when(c == 0)
        def _():
            pltpu.async_copy(q0_hbm.at[didx3.at[ci, 0]], qbuf, sq)
            pltpu.async_copy(kv0_hbm.at[sidx3.at[ci, 0]], kvbuf, skv)

        @pl.when(c == 1)
        def _():
            pltpu.async_copy(q1_hbm.at[didx3.at[ci, 0]], qbuf, sq)
            pltpu.async_copy(kv1_hbm.at[sidx3.at[ci, 0]], kvbuf, skv)

    def _wait_gather(qbuf, kvbuf, sq, skv):
        pltpu.make_async_copy(q0_hbm.at[pl.ds(0, _C)], qbuf, sq).wait()
        pltpu.make_async_copy(kv0_hbm.at[pl.ds(0, _C)], kvbuf, skv).wait()

    def _scat(ci, comb, ss):
        pltpu.async_copy(comb, acc.at[didx3.at[ci, 0]], ss, add=True)

    def _wait_scat(comb, ss):
        pltpu.make_async_copy(z_hbm.at[pl.ds(0, _C)], comb, ss).wait()

    perm8 = lanes ^ 8
    perm4 = lanes ^ 4
    perm2 = lanes ^ 2
    perm1 = lanes ^ 1
    lane_grp = lanes // 4
    bcast_idx = [jnp.full((16,), 4 * h, jnp.int32) for h in range(_HH)]

    def _perm(x, pidx):
        return x.at[pidx].get(mode="promise_in_bounds")

    def _compute(qbuf, kvbuf, comb):
        @plsc.parallel_loop(0, _C, unroll=1)
        def _edge(e):
            parts = []
            vs = []
            for hp in range(_HH // 2):
                qp = plsc.unpack(qbuf[e, pl.ds(hp * 32, 32)],
                                 format=plsc.PackFormat.INTERLEAVED)
                kp = plsc.unpack(kvbuf[e, pl.ds(hp * 32, 32)],
                                 format=plsc.PackFormat.INTERLEAVED)
                vp = plsc.unpack(kvbuf[e, pl.ds(_HD + hp * 32, 32)],
                                 format=plsc.PackFormat.INTERLEAVED)
                vs += [vp[0], vp[1]]
                for j in range(2):
                    p = qp[j] * kp[j]
                    p = p + _perm(p, perm8)
                    p = p + _perm(p, perm4)
                    parts.append(p)
            m = parts[0]
            for h in range(1, _HH):
                m = jnp.where(lane_grp == h, parts[h], m)
            m = m + _perm(m, perm2)
            m = m + _perm(m, perm1)
            u4 = jnp.clip(m * 0.25, -_CLAMP, _CLAMP)
            w4 = jnp.exp(u4)
            for h in range(_HH):
                w = _perm(w4, bcast_idx[h])
                comb[e, pl.ds(h * _DH, _DH)] = w * vs[h]
            wsel = jnp.where(lanes < _HH, _perm(w4, (lanes % 4) * 4), 0.0)
            plsc.store_scatter(comb,
                               [jnp.full((16,), e, jnp.int32), _HD + lanes],
                               wsel, mask=lanes < 8)

    _gather(0, qbuf_a, kvbuf_a, sem_qa, sem_kva)

    def _pair(i, carry):
        ca = 2 * i
        _gather(ca + 1, qbuf_b, kvbuf_b, sem_qb, sem_kvb)

        @pl.when(i > 0)
        def _():
            _wait_scat(comb_a, sem_sa)

        _wait_gather(qbuf_a, kvbuf_a, sem_qa, sem_kva)
        _compute(qbuf_a, kvbuf_a, comb_a)
        _scat(ca, comb_a, sem_sa)

        @pl.when(i < _NPAIR - 1)
        def _():
            _gather(ca + 2, qbuf_a, kvbuf_a, sem_qa, sem_kva)

        @pl.when(i > 0)
        def _():
            _wait_scat(comb_b, sem_sb)

        _wait_gather(qbuf_b, kvbuf_b, sem_qb, sem_kvb)
        _compute(qbuf_b, kvbuf_b, comb_b)
        _scat(ca + 1, comb_b, sem_sb)
        return carry

    lax.fori_loop(0, _NPAIR, _pair, 0)

    _wait_scat(comb_a, sem_sa)
    _wait_scat(comb_b, sem_sb)

    plsc.subcore_barrier()
    pltpu.sync_copy(acc.at[pl.ds(r0, _RPT)], out_hbm.at[c, pl.ds(r0, _RPT)])


def _sc_edge(src3, dst3, q0, q1, kv0, kv1, z):
    mesh = plsc.VectorSubcoreMesh(core_axis_name="c", subcore_axis_name="s")
    return pl.kernel(
        _sc_body,
        mesh=mesh,
        compiler_params=pltpu.CompilerParams(needs_layout_passes=False,
                                             use_tc_tiling_on_sc=False),
        out_type=[
            jax.ShapeDtypeStruct((2, _NP, _CW), _F32),
        ],
        scratch_types=[
            pltpu.VMEM((_NCHUNK, 1, _C), jnp.int32),
            pltpu.VMEM((_NCHUNK, 1, _C), jnp.int32),
            pltpu.VMEM((_C, _HD), jnp.bfloat16),
            pltpu.VMEM((_C, _D), jnp.bfloat16),
            pltpu.VMEM((_C, _CW), _F32),
            pltpu.VMEM((_C, _HD), jnp.bfloat16),
            pltpu.VMEM((_C, _D), jnp.bfloat16),
            pltpu.VMEM((_C, _CW), _F32),
            pltpu.VMEM_SHARED((_NP, _CW), _F32),
            pltpu.SemaphoreType.DMA,
            pltpu.SemaphoreType.DMA,
            pltpu.SemaphoreType.DMA,
            pltpu.SemaphoreType.DMA,
            pltpu.SemaphoreType.DMA,
            pltpu.SemaphoreType.DMA,
        ],
    )(src3, dst3, q0, q1, kv0, kv1, z)


def _epi_body(a_ref, feat_ref, wo_ref, g1_ref, bt1_ref, w1_ref,
              bb1_ref, w2_ref, bb2_ref, g2_ref, bt2_ref, out_ref):
    a0 = a_ref[0]
    a1 = a_ref[1]
    num = jnp.concatenate([a0[:, :_HD], a1[:, :_HD]], axis=1)
    den = jnp.concatenate([a0[:, _HD:_HD + _HH],
                           a1[:, _HD:_HD + _HH]], axis=1)
    r = lax.broadcasted_iota(jnp.int32, (_H, _D), 0)
    cc = lax.broadcasted_iota(jnp.int32, (_H, _D), 1)
    sel = (cc // _DH == r).astype(_F32)
    den_e = lax.dot_general(den, sel, (((1,), (0,)), ((), ())),
                            preferred_element_type=_F32)
    den_e = jnp.where(den_e == 0.0, 1.0, den_e)
    agg = num / den_e
    uh = lax.dot_general(agg, wo_ref[...], _DN_T, preferred_element_type=_F32)
    x1 = feat_ref[...] + uh
    mu = jnp.mean(x1, axis=-1, keepdims=True)
    var = jnp.mean((x1 - mu) ** 2, axis=-1, keepdims=True)
    h1 = (x1 - mu) / jnp.sqrt(var + 1e-5) * g1_ref[...] + bt1_ref[...]
    t = jnp.maximum(
        lax.dot_general(h1, w1_ref[...], _DN_T, preferred_element_type=_F32)
        + bb1_ref[...], 0.0)
    f = lax.dot_general(t, w2_ref[...], _DN_T,
                        preferred_element_type=_F32) + bb2_ref[...]
    x2 = h1 + f
    mu2 = jnp.mean(x2, axis=-1, keepdims=True)
    var2 = jnp.mean((x2 - mu2) ** 2, axis=-1, keepdims=True)
    out_ref[...] = (x2 - mu2) / jnp.sqrt(var2 + 1e-5) * g2_ref[...] \
        + bt2_ref[...]


def _epi(a, feat, Wo, ln1_g, ln1_b, W1, b1, W2, b2, ln2_g, ln2_b):
    full = lambda shape: pl.BlockSpec(shape, lambda i: tuple(0 for _ in shape))
    return pl.pallas_call(
        _epi_body,
        grid=(_GRID,),
        in_specs=[
            pl.BlockSpec((2, _BLK, _CW), lambda i: (0, i, 0)),
            pl.BlockSpec((_BLK, _D), lambda i: (i, 0)),
            full((_D, _D)),
            full((_D,)),
            full((_D,)),
            full((_DFF, _D)),
            full((_DFF,)),
            full((_D, _DFF)),
            full((_D,)),
            full((_D,)),
            full((_D,)),
        ],
        out_specs=pl.BlockSpec((_BLK, _D), lambda i: (i, 0)),
        out_shape=jax.ShapeDtypeStruct((_N, _D), _F32),
    )(a, feat, Wo, ln1_g, ln1_b, W1, b1, W2, b2, ln2_g, ln2_b)


def kernel(feat, edge_index, Wq, Wk, Wv, Wo, ln1_g, ln1_b, W1, b1, W2, b2,
           ln2_g, ln2_b):
    src3 = edge_index[0].reshape(_E // _C, 1, _C)
    dst3 = edge_index[1].reshape(_E // _C, 1, _C)
    q0, q1, kv0, kv1 = _proj(feat, Wq, Wk, Wv)
    z = jnp.zeros((_NP, _CW), _F32)
    (acc,) = _sc_edge(src3, dst3, q0, q1, kv0, kv1, z)
    out = _epi(acc, feat, Wo, ln1_g, ln1_b,
               W1, b1, W2, b2, ln2_g, ln2_b)
    return out
